# baseline (device time: 15739 ns/iter reference)
import jax
import jax.numpy as jnp
from jax import lax
from jax.experimental import pallas as pl
from jax.experimental.pallas import tpu as pltpu

PH = 13
ROWS = 16
H = PH * ROWS


def kernel(x):
    m, n = x.shape
    d_rows = m - 2 * H

    def body(x_ref, out_ref, comm_ref, x_send_sems, x_recv_sems,
             y_send_sems, y_recv_sems, loc_sems, chunk_sem):
        my_x = lax.axis_index("x")
        my_y = lax.axis_index("y")
        other_x = 1 - my_x
        other_y = 1 - my_y

        barrier_sem = pltpu.get_barrier_semaphore()
        for dev in [(other_x, my_y), (my_x, other_y)]:
            pl.semaphore_signal(
                barrier_sem, inc=1,
                device_id=dev, device_id_type=pl.DeviceIdType.MESH,
            )
        pl.semaphore_wait(barrier_sem, 2)

        chunk_copy = pltpu.make_async_copy(
            x_ref, out_ref.at[pl.ds(my_x * m, m), :], chunk_sem
        )
        chunk_copy.start()

        x_sends = []
        for k in range(PH):
            row = my_y * H + k * ROWS
            rdma = pltpu.make_async_remote_copy(
                src_ref=x_ref.at[pl.ds(row, ROWS), :],
                dst_ref=comm_ref.at[pl.ds(row, ROWS), :],
                send_sem=x_send_sems.at[k],
                recv_sem=x_recv_sems.at[k],
                device_id=(other_x, my_y),
                device_id_type=pl.DeviceIdType.MESH,
            )
            rdma.start()
            x_sends.append(rdma)
        d_send = pltpu.make_async_remote_copy(
            src_ref=x_ref.at[pl.ds(2 * H, d_rows), :],
            dst_ref=comm_ref.at[pl.ds(2 * H, d_rows), :],
            send_sem=x_send_sems.at[PH],
            recv_sem=x_recv_sems.at[PH],
            device_id=(other_x, my_y),
            device_id_type=pl.DeviceIdType.MESH,
        )
        d_send.start()

        y_sends = []
        loc_copies = []
        for k in range(PH):
            x_sends[k].wait_recv()
            row = my_y * H + k * ROWS
            rdma = pltpu.make_async_remote_copy(
                src_ref=comm_ref.at[pl.ds(row, ROWS), :],
                dst_ref=comm_ref.at[pl.ds(row, ROWS), :],
                send_sem=y_send_sems.at[k],
                recv_sem=y_recv_sems.at[k],
                device_id=(my_x, other_y),
                device_id_type=pl.DeviceIdType.MESH,
            )
            rdma.start()
            y_sends.append(rdma)
            loc = pltpu.make_async_copy(
                comm_ref.at[pl.ds(row, ROWS), :],
                out_ref.at[pl.ds(other_x * m + row, ROWS), :],
                loc_sems.at[k],
            )
            loc.start()
            loc_copies.append(loc)

        d_send.wait_recv()
        d_copy = pltpu.make_async_copy(
            comm_ref.at[pl.ds(2 * H, d_rows), :],
            out_ref.at[pl.ds(other_x * m + 2 * H, d_rows), :],
            loc_sems.at[2 * PH],
        )
        d_copy.start()

        for k in range(PH):
            row = other_y * H + k * ROWS
            recv = pltpu.make_async_remote_copy(
                src_ref=comm_ref.at[pl.ds(row, ROWS), :],
                dst_ref=comm_ref.at[pl.ds(row, ROWS), :],
                send_sem=y_send_sems.at[k],
                recv_sem=y_recv_sems.at[k],
                device_id=(my_x, other_y),
                device_id_type=pl.DeviceIdType.MESH,
            )
            recv.wait_recv()
            loc = pltpu.make_async_copy(
                comm_ref.at[pl.ds(row, ROWS), :],
                out_ref.at[pl.ds(other_x * m + row, ROWS), :],
                loc_sems.at[PH + k],
            )
            loc.start()
            loc_copies.append(loc)

        chunk_copy.wait()
        d_send.wait_send()
        d_copy.wait()
        for k in range(PH):
            x_sends[k].wait_send()
            y_sends[k].wait_send()
        for loc in loc_copies:
            loc.wait()

    return pl.pallas_call(
        body,
        out_shape=jax.ShapeDtypeStruct((2 * m, n), x.dtype),
        in_specs=[pl.BlockSpec(memory_space=pltpu.VMEM)],
        out_specs=pl.BlockSpec(memory_space=pltpu.VMEM),
        scratch_shapes=[
            pltpu.VMEM((m, n), x.dtype),
            pltpu.SemaphoreType.DMA((PH + 1,)),
            pltpu.SemaphoreType.DMA((PH + 1,)),
            pltpu.SemaphoreType.DMA((PH,)),
            pltpu.SemaphoreType.DMA((PH,)),
            pltpu.SemaphoreType.DMA((2 * PH + 1,)),
            pltpu.SemaphoreType.DMA,
        ],
        compiler_params=pltpu.CompilerParams(collective_id=0),
    )(x)
